# baseline (speedup 1.0000x reference)
import functools

import jax
import jax.numpy as jnp
from jax import lax
from jax.experimental import pallas as pl
from jax.experimental.pallas import tpu as pltpu
from jax.experimental.pallas import tpu_sc as plsc

_CHUNK = 16


@functools.cache
def _sc_embed(n, d, dtype):
    info = plsc.get_sparse_core_info()
    nw = info.num_cores * info.num_subcores
    rpw = n // nw
    nchunks = rpw // _CHUNK
    mesh = plsc.VectorSubcoreMesh(core_axis_name="c", subcore_axis_name="s")

    @functools.partial(
        pl.kernel,
        mesh=mesh,
        out_type=jax.ShapeDtypeStruct((n, d), dtype),
        scratch_types=[
            pltpu.VMEM((16,), jnp.int32),
            pltpu.VMEM((_CHUNK,), jnp.int32),
            pltpu.VMEM((_CHUNK, d), dtype),
            pltpu.SemaphoreType.DMA,
        ],
    )
    def k(s_hbm, w_hbm, out_hbm, s_v, idx_v, buf_v, sem):
        wid = lax.axis_index("s") * info.num_cores + lax.axis_index("c")
        base = wid * rpw
        pltpu.sync_copy(s_hbm, s_v)
        s = s_v[...][0]

        @pl.when(base + rpw <= s)
        def _identity():
            pltpu.sync_copy(
                w_hbm.at[pl.ds(base, rpw)], out_hbm.at[pl.ds(base, rpw)]
            )

        @pl.when(base + rpw > s)
        def _clamped():
            def body(c, carry):
                cb = base + c * _CHUNK
                idx_v[...] = jnp.minimum(
                    cb + lax.iota(jnp.int32, _CHUNK), s - 1
                )
                pltpu.async_copy(w_hbm.at[idx_v], buf_v, sem).wait()
                pltpu.sync_copy(buf_v, out_hbm.at[pl.ds(cb, _CHUNK)])
                return carry

            lax.fori_loop(0, nchunks, body, 0)

    return k


def kernel(seq_len, weight):
    n, d = weight.shape
    s_arr = jnp.full((16,), seq_len, jnp.int32)
    return _sc_embed(n, d, weight.dtype)(s_arr, weight)


# speedup vs baseline: 36.0296x; 36.0296x over previous
import functools

import jax
import jax.numpy as jnp
from jax import lax
from jax.experimental import pallas as pl
from jax.experimental.pallas import tpu as pltpu
from jax.experimental.pallas import tpu_sc as plsc

_CHUNK = 8
_GCHUNK = 16


@functools.cache
def _sc_embed(n, d, dtype):
    info = plsc.get_sparse_core_info()
    nw = info.num_cores * info.num_subcores
    rpw = n // nw
    nchunks = rpw // _CHUNK
    ngchunks = rpw // _GCHUNK
    mesh = plsc.VectorSubcoreMesh(core_axis_name="c", subcore_axis_name="s")

    @functools.partial(
        pl.kernel,
        mesh=mesh,
        out_type=jax.ShapeDtypeStruct((n, d), dtype),
        scratch_types=[
            pltpu.VMEM((16,), jnp.int32),
            pltpu.VMEM((_GCHUNK,), jnp.int32),
            pltpu.VMEM((_GCHUNK, d), dtype),
            pltpu.SemaphoreType.DMA,
            pltpu.SemaphoreType.DMA,
            pltpu.SemaphoreType.DMA,
            pltpu.SemaphoreType.DMA,
        ],
    )
    def k(s_hbm, w_hbm, out_hbm, s_v, idx_v, buf_v, r0, r1, w0, w1):
        wid = lax.axis_index("s") * info.num_cores + lax.axis_index("c")
        base = wid * rpw
        pltpu.sync_copy(s_hbm, s_v)
        s = s_v[...][0]
        rsem = (r0, r1)
        wsem = (w0, w1)

        @pl.when(base + rpw <= s)
        def _identity():
            def read(c):
                slot = c & 1
                return pltpu.async_copy(
                    w_hbm.at[pl.ds(base + c * _CHUNK, _CHUNK)],
                    buf_v.at[pl.ds(slot * _CHUNK, _CHUNK)],
                    rsem[slot],
                )

            def write(c):
                slot = c & 1
                return pltpu.async_copy(
                    buf_v.at[pl.ds(slot * _CHUNK, _CHUNK)],
                    out_hbm.at[pl.ds(base + c * _CHUNK, _CHUNK)],
                    wsem[slot],
                )

            hr = [None, None]
            hw = [None, None]
            hr[0] = read(0)
            for c in range(nchunks):
                slot = c & 1
                nxt = slot ^ 1
                if c + 1 < nchunks:
                    if hw[nxt] is not None:
                        hw[nxt].wait()
                    hr[nxt] = read(c + 1)
                hr[slot].wait()
                hw[slot] = write(c)
            hw[(nchunks - 1) & 1].wait()
            if nchunks > 1:
                hw[nchunks & 1].wait()

        @pl.when(base + rpw > s)
        def _clamped():
            def body(c, carry):
                cb = base + c * _GCHUNK
                idx_v[...] = jnp.minimum(
                    cb + lax.iota(jnp.int32, _GCHUNK), s - 1
                )
                pltpu.async_copy(w_hbm.at[idx_v], buf_v, r0).wait()
                pltpu.sync_copy(buf_v, out_hbm.at[pl.ds(cb, _GCHUNK)])
                return carry

            lax.fori_loop(0, ngchunks, body, 0)

    return k


def kernel(seq_len, weight):
    n, d = weight.shape
    s_arr = jnp.full((16,), seq_len, jnp.int32)
    return _sc_embed(n, d, weight.dtype)(s_arr, weight)
